# baseline (speedup 1.0000x reference)
import functools

import jax
import jax.numpy as jnp
from jax import lax
from jax.experimental import pallas as pl
from jax.experimental.pallas import tpu as pltpu
from jax.experimental.pallas import tpu_sc as plsc

NC = 2
NS = 16
NW = NC * NS


def _build_gather(batch: int, hist: int, emb_dim: int):
    assert batch % NW == 0
    bpw = batch // NW
    pair = 2
    assert hist % pair == 0
    n_steps = hist // pair
    nbuf = 3
    ahead = 2
    assert (n_steps - 1) % nbuf == 0

    mesh = plsc.VectorSubcoreMesh(core_axis_name="c", subcore_axis_name="s")

    @functools.partial(
        pl.kernel,
        out_type=jax.ShapeDtypeStruct((hist, batch, emb_dim), jnp.float32),
        mesh=mesh,
        compiler_params=pltpu.CompilerParams(use_tc_tiling_on_sc=True),
        scratch_types=[
            pltpu.VMEM((1, hist * bpw), jnp.int32),
            [pltpu.VMEM((pair * bpw, emb_dim), jnp.float32) for _ in range(nbuf)],
            [pltpu.SemaphoreType.DMA for _ in range(nbuf)],
            [pltpu.SemaphoreType.DMA for _ in range(nbuf)],
        ],
    )
    def gather_kernel(ids_hbm, table_hbm, out_hbm, idx_v, bufs, gsems, wsems):
        w = lax.axis_index("s") * NC + lax.axis_index("c")
        pltpu.sync_copy(ids_hbm.at[w], idx_v)

        def fire_gather(p, b):
            pltpu.async_copy(
                table_hbm.at[idx_v.at[0, pl.ds(p * pair * bpw, pair * bpw)]],
                bufs[b], gsems[b])

        def wait_gather(p, b):
            pltpu.make_async_copy(
                table_hbm.at[idx_v.at[0, pl.ds(p * pair * bpw, pair * bpw)]],
                bufs[b], gsems[b]).wait()

        def fire_write(p, b):
            for r in range(pair):
                pltpu.async_copy(
                    bufs[b].at[pl.ds(r * bpw, bpw)],
                    out_hbm.at[p * pair + r, pl.ds(w * bpw, bpw)], wsems[b])

        def wait_write(p, b):
            for r in range(pair):
                pltpu.make_async_copy(
                    bufs[b].at[pl.ds(r * bpw, bpw)],
                    out_hbm.at[p * pair + r, pl.ds(w * bpw, bpw)], wsems[b]).wait()

        for p in range(ahead):
            fire_gather(p, p % nbuf)
        wait_gather(0, 0)
        fire_write(0, 0)
        fire_gather(ahead, ahead % nbuf)

        def step(k, carry):
            for i in range(nbuf):
                p = 1 + k * nbuf + i
                b = (1 + i) % nbuf
                wait_gather(p, b)
                fire_write(p, b)
                bn = (b + ahead) % nbuf
                wait_write(p - (nbuf - ahead), bn)

                @pl.when(p + ahead < n_steps)
                def _():
                    fire_gather(p + ahead, bn)
            return carry

        lax.fori_loop(0, (n_steps - 1) // nbuf, step, 0)

        for p in range(n_steps - nbuf + ahead, n_steps):
            wait_write(p, p % nbuf)

    return gather_kernel


def kernel(ids, table):
    b, l = ids.shape
    bpw = b // NW
    ids_flat = (ids.T.astype(jnp.int32)
                .reshape(l, NW, bpw).transpose(1, 0, 2).reshape(NW, 1, l * bpw))
    out_t = _build_gather(b, l, table.shape[1])(ids_flat, table)
    return out_t.transpose(1, 0, 2)
